# baseline (speedup 1.0000x reference)
import functools

import jax
import jax.numpy as jnp
from jax import lax
from jax.experimental import pallas as pl
from jax.experimental.pallas import tpu as pltpu
from jax.experimental.pallas import tpu_sc as plsc

N1 = 2048
N2 = 8192
F = 512
BY = 512
NBLK = N2 // BY
B24 = 24


def _pack4(c0, c1, c2, c3):
    return ((c0 * B24 + c1) * B24 + c2) * B24 + c3


def _match_body(x_ref, xt_ref, yt_ref, sel_ref, fct_ref):
    j = pl.program_id(0)
    kx = _pack4(x_ref[:, 0:1], x_ref[:, 1:2], x_ref[:, 2:3], x_ref[:, 3:4])
    yb = yt_ref[:, pl.ds(j * BY, BY)]
    q = yb >> 3
    ky = _pack4(q[0:1, :], q[1:2, :], q[2:3, :], q[3:4, :])
    ii = lax.broadcasted_iota(jnp.int32, (N1, BY), 0)
    val = jnp.where(kx == ky, ii, N1)
    m = jnp.min(val, axis=0)
    matched = m < N1
    sel_ref[pl.ds(j * BY, BY)] = jnp.where(matched, m, 0)
    x0 = xt_ref[:, 0:1].astype(jnp.float32)
    fct_ref[:, pl.ds(j * BY, BY)] = jnp.where(
        matched[None, :], q.astype(jnp.float32), x0)


def _match(x, xt, yt):
    return pl.pallas_call(
        _match_body,
        grid=(NBLK,),
        in_specs=[
            pl.BlockSpec((N1, 4), lambda j: (0, 0)),
            pl.BlockSpec((4, N1), lambda j: (0, 0)),
            pl.BlockSpec((4, N2), lambda j: (0, 0)),
        ],
        out_specs=[
            pl.BlockSpec((N2,), lambda j: (0,)),
            pl.BlockSpec((4, N2), lambda j: (0, 0)),
        ],
        out_shape=[
            jax.ShapeDtypeStruct((N2,), jnp.int32),
            jax.ShapeDtypeStruct((4, N2), jnp.float32),
        ],
    )(x, xt, yt)


_NC = 2
_NS = 16
_NW = _NC * _NS
_BPW = N2 // _NW
_CH = 128


@functools.cache
def _gather_feats_kernel():
    @functools.partial(
        pl.kernel,
        mesh=plsc.VectorSubcoreMesh(core_axis_name="c", subcore_axis_name="s"),
        out_type=jax.ShapeDtypeStruct((N2, F), jnp.float32),
        scratch_types=[
            pltpu.VMEM((_BPW,), jnp.int32),
            pltpu.VMEM((_CH, F), jnp.float32),
            pltpu.SemaphoreType.DMA,
        ],
    )
    def _gather_feats(feat_hbm, sel_hbm, out_hbm, idx_v, rows_v, sem):
        wid = lax.axis_index("s") * _NC + lax.axis_index("c")
        base = wid * _BPW
        pltpu.sync_copy(sel_hbm.at[pl.ds(base, _BPW)], idx_v)
        for c in range(_BPW // _CH):
            pltpu.async_copy(
                feat_hbm.at[idx_v.at[pl.ds(c * _CH, _CH)]], rows_v, sem).wait()
            pltpu.sync_copy(rows_v, out_hbm.at[pl.ds(base + c * _CH, _CH)])

    return _gather_feats


def kernel(x_features, x_coords, y_coords):
    x = x_coords.astype(jnp.int32)
    y = y_coords.astype(jnp.int32)
    sel, fct = _match(x, x.T, y.T)
    feats = _gather_feats_kernel()(x_features, sel)
    return fct.T, feats
